# Initial kernel scaffold; baseline (speedup 1.0000x reference)
import jax
import jax.numpy as jnp
from jax.experimental import pallas as pl


def kernel(x, adj, W1, a1_src, a1_dst, b1, W2, a2_src, a2_dst, b2, W3, a3_src, a3_dst, b3):
    raise NotImplementedError("write your pallas kernel here")


# speedup vs baseline: 5.6810x; 5.6810x over previous
import functools

import jax
import jax.numpy as jnp
from jax import lax
from jax.experimental import pallas as pl
from jax.experimental.pallas import tpu as pltpu
from jax.experimental.pallas import tpu_sc as plsc

N_NODES = 10000
N_EDGES = 160000
NP = 10240
EP = 163840
HP = 16
L = 16
C = 128
NC, NS = 2, 16
F32 = jnp.float32

_SC_PARAMS = pltpu.CompilerParams(needs_layout_passes=False,
                                  use_tc_tiling_on_sc=False)


def _dot(a, b):
    return lax.dot_general(a, b, (((1,), (0,)), ((), ())),
                           preferred_element_type=F32,
                           precision=lax.Precision.HIGHEST)


def _tc_layer(KG, G, apply_act):
    BN = 512
    K = KG * 64

    def body(x_ref, w_ref, b_ref, asm_ref, adm_ref, h_ref, asrc_ref, adst_ref):
        xfull = jnp.concatenate([x_ref[k] for k in range(KG)], axis=1)
        if apply_act:
            xfull = jnp.maximum(xfull + b_ref[...], 0.0)
        hfull = _dot(xfull, w_ref[...])
        for g in range(G):
            h_ref[g] = hfull[:, g * 64:(g + 1) * 64]
        asrc_ref[...] = _dot(hfull, asm_ref[...])
        adst_ref[...] = _dot(hfull, adm_ref[...])

    return pl.pallas_call(
        body,
        grid=(NP // BN,),
        in_specs=[
            pl.BlockSpec((KG, BN, 64), lambda i: (0, i, 0)),
            pl.BlockSpec((K, G * 64), lambda i: (0, 0)),
            pl.BlockSpec((1, K), lambda i: (0, 0)),
            pl.BlockSpec((G * 64, HP), lambda i: (0, 0)),
            pl.BlockSpec((G * 64, HP), lambda i: (0, 0)),
        ],
        out_specs=[
            pl.BlockSpec((G, BN, 64), lambda i: (0, i, 0)),
            pl.BlockSpec((BN, HP), lambda i: (i, 0)),
            pl.BlockSpec((BN, HP), lambda i: (i, 0)),
        ],
        out_shape=[
            jax.ShapeDtypeStruct((G, NP, 64), F32),
            jax.ShapeDtypeStruct((NP, HP), F32),
            jax.ShapeDtypeStruct((NP, HP), F32),
        ],
    )


def _sc_edge_softmax():
    mesh = plsc.VectorSubcoreMesh(core_axis_name="c", subcore_axis_name="s")
    EW = EP // (NC * NS)
    KCH = EW // C
    RPT = NP // NS

    @functools.partial(
        pl.kernel, mesh=mesh, compiler_params=_SC_PARAMS,
        out_type=[
            jax.ShapeDtypeStruct((EP, HP), F32),
            jax.ShapeDtypeStruct((NP, HP), F32),
            jax.ShapeDtypeStruct((NP, HP), F32),
        ],
        scratch_types=[
            pltpu.VMEM((C,), jnp.int32),
            pltpu.VMEM((C,), jnp.int32),
            pltpu.VMEM((C, HP), F32),
            pltpu.VMEM((C, HP), F32),
            pltpu.VMEM((C, HP), F32),
            pltpu.VMEM_SHARED((NP, HP), F32),
            pltpu.SemaphoreType.DMA,
        ],
    )
    def ka(src_hbm, dst_hbm, asrc_hbm, adst_hbm, ex_hbm, d0_hbm, d1_hbm,
           sidx, didx, asr, adr, exv, den_sh, sem):
        cid = lax.axis_index("c")
        sid = lax.axis_index("s")
        wid = sid * NC + cid

        def zrow(r, _):
            exv[r, :] = jnp.zeros((L,), F32)
            return 0
        lax.fori_loop(0, C, zrow, 0)

        def zslice(r, _):
            pltpu.sync_copy(exv, den_sh.at[pl.ds(sid * RPT + r * C, C)])
            return 0
        lax.fori_loop(0, RPT // C, zslice, 0)
        plsc.subcore_barrier()

        def chunk(k, _):
            base = wid * EW + k * C
            pltpu.sync_copy(src_hbm.at[pl.ds(base, C)], sidx)
            pltpu.sync_copy(dst_hbm.at[pl.ds(base, C)], didx)
            cp_a = pltpu.async_copy(asrc_hbm.at[sidx], asr, sem)
            cp_b = pltpu.async_copy(adst_hbm.at[didx], adr, sem)
            cp_a.wait()
            cp_b.wait()

            def edge(e, _):
                v = asr[e, :] + adr[e, :]
                v = jnp.where(v > 0, v, 0.2 * v)
                exv[e, :] = jnp.exp(v)
                return 0
            lax.fori_loop(0, C, edge, 0)
            pltpu.sync_copy(exv, ex_hbm.at[pl.ds(base, C)])
            pltpu.sync_copy(exv, den_sh.at[didx], add=True)
            return 0
        lax.fori_loop(0, KCH, chunk, 0)
        plsc.subcore_barrier()

        off = sid * RPT

        @pl.when(cid == 0)
        def _():
            pltpu.sync_copy(den_sh.at[pl.ds(off, RPT)], d0_hbm.at[pl.ds(off, RPT)])

        @pl.when(cid == 1)
        def _():
            pltpu.sync_copy(den_sh.at[pl.ds(off, RPT)], d1_hbm.at[pl.ds(off, RPT)])

    return ka


def _sc_aggregate():
    mesh = plsc.VectorSubcoreMesh(core_axis_name="c", subcore_axis_name="s")
    G, DG = 16, 64
    ET = EP // NS
    KCH = ET // C
    RPT = NP // NS

    @functools.partial(
        pl.kernel, mesh=mesh, compiler_params=_SC_PARAMS,
        out_type=[
            jax.ShapeDtypeStruct((G, NP, DG), F32),
            jax.ShapeDtypeStruct((NP, HP), F32),
            jax.ShapeDtypeStruct((NP, HP), F32),
        ],
        scratch_types=[
            pltpu.VMEM((C,), jnp.int32),
            pltpu.VMEM((C,), jnp.int32),
            pltpu.VMEM((C, DG), F32),
            pltpu.VMEM((C, HP), F32),
            pltpu.VMEM((C, HP), F32),
            pltpu.VMEM_SHARED((NP, DG), F32),
            pltpu.SemaphoreType.DMA,
        ],
    )
    def kb(src_hbm, dst_hbm, ex_hbm, d0_hbm, d1_hbm, h_hbm,
           agg_hbm, rec0_hbm, rec1_hbm,
           sidx, didx, hrows, exr, recr, acc_sh, sem):
        cid = lax.axis_index("c")
        sid = lax.axis_index("s")
        lanes = lax.iota(jnp.int32, L)

        def recchunk(r, _):
            roff = sid * RPT + r * C
            pltpu.sync_copy(d0_hbm.at[pl.ds(roff, C)], exr)
            pltpu.sync_copy(d1_hbm.at[pl.ds(roff, C)], recr)

            def rrow(q, _):
                recr[q, :] = 1.0 / (exr[q, :] + recr[q, :])
                return 0
            lax.fori_loop(0, C, rrow, 0)

            @pl.when(cid == 0)
            def _():
                pltpu.sync_copy(recr, rec0_hbm.at[pl.ds(roff, C)])

            @pl.when(cid == 1)
            def _():
                pltpu.sync_copy(recr, rec1_hbm.at[pl.ds(roff, C)])
            return 0
        lax.fori_loop(0, RPT // C, recchunk, 0)

        def zero_hrows():
            def zr(r, _):
                for j in range(DG // L):
                    hrows[r, pl.ds(j * L, L)] = jnp.zeros((L,), F32)
                return 0
            lax.fori_loop(0, C, zr, 0)

        def process(g, rec_hbm):
            oh = jnp.where(lanes == g, 1.0, 0.0).astype(F32)

            def chunk(k, _):
                base = sid * ET + k * C
                pltpu.sync_copy(src_hbm.at[pl.ds(base, C)], sidx)
                pltpu.sync_copy(dst_hbm.at[pl.ds(base, C)], didx)
                cp_h = pltpu.async_copy(h_hbm.at[g].at[sidx], hrows, sem)
                cp_e = pltpu.async_copy(ex_hbm.at[pl.ds(base, C)], exr, sem)
                cp_r = pltpu.async_copy(rec_hbm.at[didx], recr, sem)
                cp_h.wait()
                cp_e.wait()
                cp_r.wait()

                def edge(e, _):
                    att = exr[e, :] * recr[e, :]
                    a0 = jnp.full((L,), jnp.sum(att * oh), F32)
                    for j in range(DG // L):
                        hrows[e, pl.ds(j * L, L)] = hrows[e, pl.ds(j * L, L)] * a0
                    return 0
                lax.fori_loop(0, C, edge, 0)
                pltpu.sync_copy(hrows, acc_sh.at[didx], add=True)
                return 0
            lax.fori_loop(0, KCH, chunk, 0)

        for gi in range(G // NC):
            zero_hrows()

            def zs(r, _):
                pltpu.sync_copy(hrows, acc_sh.at[pl.ds(sid * RPT + r * C, C)])
                return 0
            lax.fori_loop(0, RPT // C, zs, 0)
            plsc.subcore_barrier()

            @pl.when(cid == 0)
            def _():
                process(2 * gi, rec0_hbm)

            @pl.when(cid == 1)
            def _():
                process(2 * gi + 1, rec1_hbm)

            plsc.subcore_barrier()
            off = sid * RPT

            @pl.when(cid == 0)
            def _():
                pltpu.sync_copy(acc_sh.at[pl.ds(off, RPT)],
                                agg_hbm.at[2 * gi, pl.ds(off, RPT)])

            @pl.when(cid == 1)
            def _():
                pltpu.sync_copy(acc_sh.at[pl.ds(off, RPT)],
                                agg_hbm.at[2 * gi + 1, pl.ds(off, RPT)])

            plsc.subcore_barrier()

    return kb


def _sc_aggregate3():
    mesh = plsc.VectorSubcoreMesh(core_axis_name="c", subcore_axis_name="s")
    DG = 64
    EW = EP // (NC * NS)
    KCH = EW // C
    RPT = NP // NS

    @functools.partial(
        pl.kernel, mesh=mesh, compiler_params=_SC_PARAMS,
        out_type=[
            jax.ShapeDtypeStruct((NP, DG), F32),
            jax.ShapeDtypeStruct((NP, DG), F32),
            jax.ShapeDtypeStruct((NP, HP), F32),
            jax.ShapeDtypeStruct((NP, HP), F32),
        ],
        scratch_types=[
            pltpu.VMEM((C,), jnp.int32),
            pltpu.VMEM((C,), jnp.int32),
            pltpu.VMEM((C, DG), F32),
            pltpu.VMEM((C, HP), F32),
            pltpu.VMEM((C, HP), F32),
            pltpu.VMEM_SHARED((NP, DG), F32),
            pltpu.SemaphoreType.DMA,
        ],
    )
    def kb3(src_hbm, dst_hbm, ex_hbm, d0_hbm, d1_hbm, h_hbm,
            p0_hbm, p1_hbm, rec0_hbm, rec1_hbm,
            sidx, didx, hrows, exr, recr, acc_sh, sem):
        cid = lax.axis_index("c")
        sid = lax.axis_index("s")
        wid = sid * NC + cid
        lanes = lax.iota(jnp.int32, L)
        oh = jnp.where(lanes == 0, 1.0, 0.0).astype(F32)

        def recchunk(r, _):
            roff = sid * RPT + r * C
            pltpu.sync_copy(d0_hbm.at[pl.ds(roff, C)], exr)
            pltpu.sync_copy(d1_hbm.at[pl.ds(roff, C)], recr)

            def rrow(q, _):
                recr[q, :] = 1.0 / (exr[q, :] + recr[q, :])
                return 0
            lax.fori_loop(0, C, rrow, 0)

            @pl.when(cid == 0)
            def _():
                pltpu.sync_copy(recr, rec0_hbm.at[pl.ds(roff, C)])

            @pl.when(cid == 1)
            def _():
                pltpu.sync_copy(recr, rec1_hbm.at[pl.ds(roff, C)])
            return 0
        lax.fori_loop(0, RPT // C, recchunk, 0)

        def zr(r, _):
            for j in range(DG // L):
                hrows[r, pl.ds(j * L, L)] = jnp.zeros((L,), F32)
            return 0
        lax.fori_loop(0, C, zr, 0)

        def zs(r, _):
            pltpu.sync_copy(hrows, acc_sh.at[pl.ds(sid * RPT + r * C, C)])
            return 0
        lax.fori_loop(0, RPT // C, zs, 0)
        plsc.subcore_barrier()

        def chunk(k, _):
            base = wid * EW + k * C
            pltpu.sync_copy(src_hbm.at[pl.ds(base, C)], sidx)
            pltpu.sync_copy(dst_hbm.at[pl.ds(base, C)], didx)
            cp_h = pltpu.async_copy(h_hbm.at[0].at[sidx], hrows, sem)
            cp_e = pltpu.async_copy(ex_hbm.at[pl.ds(base, C)], exr, sem)
            cp_h.wait()
            cp_e.wait()

            @pl.when(cid == 0)
            def _():
                pltpu.async_copy(rec0_hbm.at[didx], recr, sem).wait()

            @pl.when(cid == 1)
            def _():
                pltpu.async_copy(rec1_hbm.at[didx], recr, sem).wait()

            def edge(e, _):
                att = exr[e, :] * recr[e, :]
                a0 = jnp.full((L,), jnp.sum(att * oh), F32)
                for j in range(DG // L):
                    hrows[e, pl.ds(j * L, L)] = hrows[e, pl.ds(j * L, L)] * a0
                return 0
            lax.fori_loop(0, C, edge, 0)
            pltpu.sync_copy(hrows, acc_sh.at[didx], add=True)
            return 0
        lax.fori_loop(0, KCH, chunk, 0)
        plsc.subcore_barrier()

        off = sid * RPT

        @pl.when(cid == 0)
        def _():
            pltpu.sync_copy(acc_sh.at[pl.ds(off, RPT)], p0_hbm.at[pl.ds(off, RPT)])

        @pl.when(cid == 1)
        def _():
            pltpu.sync_copy(acc_sh.at[pl.ds(off, RPT)], p1_hbm.at[pl.ds(off, RPT)])

    return kb3


def _tc_final():
    BN = 512

    def body(p0_ref, p1_ref, b3_ref, out_ref):
        xs = p0_ref[...] + p1_ref[...] + b3_ref[...]
        xs = xs[:, :40]
        m = jnp.max(xs, axis=1, keepdims=True)
        ex = jnp.exp(xs - m)
        lse = jnp.log(jnp.sum(ex, axis=1, keepdims=True))
        out_ref[...] = xs - m - lse

    return pl.pallas_call(
        body,
        grid=(NP // BN,),
        in_specs=[
            pl.BlockSpec((BN, 64), lambda i: (i, 0)),
            pl.BlockSpec((BN, 64), lambda i: (i, 0)),
            pl.BlockSpec((1, 64), lambda i: (0, 0)),
        ],
        out_specs=pl.BlockSpec((BN, 40), lambda i: (i, 0)),
        out_shape=jax.ShapeDtypeStruct((NP, 40), F32),
    )


def _amat(a):
    m = jnp.zeros((HP, 64, HP), F32)
    m = m.at[jnp.arange(15), :, jnp.arange(15)].set(a.astype(F32))
    return m.reshape(HP * 64, HP)


def _amat3(a):
    m = jnp.zeros((64, HP), F32)
    m = m.at[:40, 0].set(a[0].astype(F32))
    return m


def kernel(x, adj, W1, a1_src, a1_dst, b1, W2, a2_src, a2_dst, b2,
           W3, a3_src, a3_dst, b3):
    src = adj[0].astype(jnp.int32)
    dst = adj[1].astype(jnp.int32)
    epad = jnp.full((EP - N_EDGES,), N_NODES, jnp.int32)
    src_p = jnp.concatenate([src, epad])
    dst_p = jnp.concatenate([dst, epad])

    xp = jnp.zeros((NP, 256), F32).at[:N_NODES].set(x.astype(F32))
    xg1 = xp.reshape(NP, 4, 64).transpose(1, 0, 2)

    W1p = jnp.zeros((256, 1024), F32).at[:, :960].set(W1)
    W2p = jnp.zeros((1024, 1024), F32).at[:960, :960].set(W2)
    W3p = jnp.zeros((1024, 64), F32).at[:960, :40].set(W3)
    b0p = jnp.zeros((1, 256), F32)
    b1p = jnp.zeros((1, 1024), F32).at[0, :960].set(b1)
    b2p = jnp.zeros((1, 1024), F32).at[0, :960].set(b2)
    b3p = jnp.zeros((1, 64), F32).at[0, :40].set(b3)

    As1, Ad1 = _amat(a1_src), _amat(a1_dst)
    As2, Ad2 = _amat(a2_src), _amat(a2_dst)
    As3, Ad3 = _amat3(a3_src), _amat3(a3_dst)

    tc1 = _tc_layer(4, 16, apply_act=False)
    tc2 = _tc_layer(16, 16, apply_act=True)
    tc3 = _tc_layer(16, 1, apply_act=True)
    ka = _sc_edge_softmax()
    kb = _sc_aggregate()
    kb3 = _sc_aggregate3()
    fin = _tc_final()

    h1, as1, ad1 = tc1(xg1, W1p, b0p, As1, Ad1)
    ex1, da1, db1 = ka(src_p, dst_p, as1, ad1)
    agg1 = kb(src_p, dst_p, ex1, da1, db1, h1)[0]

    h2, as2, ad2 = tc2(agg1, W2p, b1p, As2, Ad2)
    ex2, da2, db2 = ka(src_p, dst_p, as2, ad2)
    agg2 = kb(src_p, dst_p, ex2, da2, db2, h2)[0]

    h3, as3, ad3 = tc3(agg2, W3p, b2p, As3, Ad3)
    ex3, da3, db3 = ka(src_p, dst_p, as3, ad3)
    p0, p1 = kb3(src_p, dst_p, ex3, da3, db3, h3)[:2]

    out = fin(p0, p1, b3p)
    return out[:N_NODES]
